# baseline (speedup 1.0000x reference)
import jax
import jax.numpy as jnp
from jax.experimental import pallas as pl

_R = (0, 46, 167, 100, 191, 220, 0, 10)
_G = (160, 141, 0, 62, 30, 87, 166, 91)
_B = (177, 239, 174, 191, 75, 46, 0, 196)
_PACKED_LUT = tuple(r | (g << 8) | (b << 16) for r, g, b in zip(_R, _G, _B))

_TILE = 2048


def _lsh_kernel(x_ref, rot_ref, r_ref, g_ref, b_ref):
    v = jax.lax.dot_general(
        x_ref[...], rot_ref[...],
        dimension_numbers=(((0,), (0,)), ((), ())),
        preferred_element_type=jnp.float32,
    )
    best = v[:, 0:16]
    packed = jnp.full(best.shape, _PACKED_LUT[0], dtype=jnp.int32)
    for k in range(1, 8):
        s = v[:, 16 * k:16 * (k + 1)]
        gt = s > best
        packed = jnp.where(gt, _PACKED_LUT[k], packed)
        best = jnp.maximum(best, s)
    r_ref[...] = (packed & 0xFF).astype(jnp.uint8)
    g_ref[...] = ((packed >> 8) & 0xFF).astype(jnp.uint8)
    b_ref[...] = ((packed >> 16) & 0xFF).astype(jnp.uint8)


def kernel(inp, rotations):
    B, C, H, W = inp.shape
    HW = H * W
    n_hashes = rotations.shape[1]
    x = inp.reshape(C, HW)
    rot = jnp.transpose(rotations, (0, 2, 1)).reshape(C, 4 * n_hashes)
    rot_packed = jnp.concatenate([rot, -rot], axis=1)

    out_sds = jax.ShapeDtypeStruct((HW, n_hashes), jnp.uint8)
    r, g, b = pl.pallas_call(
        _lsh_kernel,
        grid=(HW // _TILE,),
        in_specs=[
            pl.BlockSpec((C, _TILE), lambda i: (0, i)),
            pl.BlockSpec((C, 128), lambda i: (0, 0)),
        ],
        out_specs=[pl.BlockSpec((_TILE, n_hashes), lambda i: (i, 0))] * 3,
        out_shape=[out_sds, out_sds, out_sds],
    )(x, rot_packed)
    shape = (B, H, W, n_hashes)
    return (r.reshape(shape), g.reshape(shape), b.reshape(shape))


# speedup vs baseline: 2.5682x; 2.5682x over previous
import jax
import jax.numpy as jnp
from jax.experimental import pallas as pl

_R = (0, 46, 167, 100, 191, 220, 0, 10)
_G = (160, 141, 0, 62, 30, 87, 166, 91)
_B = (177, 239, 174, 191, 75, 46, 0, 196)
_PACKED_LUT = tuple(r | (g << 8) | (b << 16) for r, g, b in zip(_R, _G, _B))

_TILE = 2048


def _lsh_kernel(rot_ref, x_ref, r_ref, g_ref, b_ref):
    v = jax.lax.dot_general(
        rot_ref[...], x_ref[...],
        dimension_numbers=(((1,), (0,)), ((), ())),
        preferred_element_type=jnp.float32,
    )
    best = v[0:16, :]
    packed = jnp.full(best.shape, _PACKED_LUT[0], dtype=jnp.int32)
    for k in range(1, 8):
        s = v[16 * k:16 * (k + 1), :]
        gt = s > best
        packed = jnp.where(gt, _PACKED_LUT[k], packed)
        best = jnp.maximum(best, s)
    r_ref[...] = (packed & 0xFF).astype(jnp.uint8)
    g_ref[...] = ((packed >> 8) & 0xFF).astype(jnp.uint8)
    b_ref[...] = ((packed >> 16) & 0xFF).astype(jnp.uint8)


def kernel(inp, rotations):
    B, C, H, W = inp.shape
    HW = H * W
    n_hashes = rotations.shape[1]
    x = inp.reshape(C, HW)
    rot = jnp.transpose(rotations, (0, 2, 1)).reshape(C, 4 * n_hashes)
    rot_packed = jnp.concatenate([rot, -rot], axis=1).T

    out_sds = jax.ShapeDtypeStruct((n_hashes, HW), jnp.uint8)
    r, g, b = pl.pallas_call(
        _lsh_kernel,
        grid=(HW // _TILE,),
        in_specs=[
            pl.BlockSpec((128, C), lambda i: (0, 0)),
            pl.BlockSpec((C, _TILE), lambda i: (0, i)),
        ],
        out_specs=[pl.BlockSpec((n_hashes, _TILE), lambda i: (0, i))] * 3,
        out_shape=[out_sds, out_sds, out_sds],
    )(rot_packed, x)
    shape = (B, H, W, n_hashes)
    return (
        r.T.reshape(shape),
        g.T.reshape(shape),
        b.T.reshape(shape),
    )
